# baseline (speedup 1.0000x reference)
import functools

import numpy as np

import jax
import jax.numpy as jnp
from jax import lax
from jax.experimental import pallas as pl
from jax.experimental.pallas import tpu as pltpu
from jax.experimental.pallas import tpu_sc as plsc

_N = 10000
_D = 128
_H = 128
_NC = 2
_NS = 16
_NW = _NC * _NS
_E = 320000
_EPW = _E // _NW
_CH = 40
_NCH = _EPW // _CH
_SLAB = 50
_NSLAB = _NCH // _SLAB
_LANES = 16
_SCALE = 1.0 / float(_H) ** 0.5
_SQ = 32768.0
_SKV = 4096.0
_W = _D // 2


def _quant_pack(x, scale, lim):
    xi = jnp.round(jnp.clip(x, -lim, lim) * scale).astype(jnp.int32)
    lo = xi[:, :_W] & 0xFFFF
    hi = xi[:, _W:] << 16
    return lo | hi


def _qkv_body(x_ref, m_ref, wq_ref, bq_ref, wk_ref, bk_ref, wv_ref, bv_ref,
              q_ref, k_ref, v_ref):
    x = x_ref[...]
    m = m_ref[...]
    q = jnp.dot(x, wq_ref[...], preferred_element_type=jnp.float32) + bq_ref[...]
    q_ref[...] = jax.nn.gelu(q) * _SCALE
    k = jnp.dot(m, wk_ref[...], preferred_element_type=jnp.float32) + bk_ref[...]
    v = jnp.dot(m, wv_ref[...], preferred_element_type=jnp.float32) + bv_ref[...]
    k_ref[...] = jax.nn.gelu(k)
    v_ref[...] = jax.nn.gelu(v)


def _qkv(query, memory, Wq, bq, Wk, bk, Wv, bv):
    blk = 1000
    return pl.pallas_call(
        _qkv_body,
        grid=(_N // blk,),
        in_specs=[
            pl.BlockSpec((blk, _D), lambda i: (i, 0)),
            pl.BlockSpec((blk, _D), lambda i: (i, 0)),
            pl.BlockSpec((_D, _H), lambda i: (0, 0)),
            pl.BlockSpec((1, _H), lambda i: (0, 0)),
            pl.BlockSpec((_D, _H), lambda i: (0, 0)),
            pl.BlockSpec((1, _H), lambda i: (0, 0)),
            pl.BlockSpec((_D, _H), lambda i: (0, 0)),
            pl.BlockSpec((1, _H), lambda i: (0, 0)),
        ],
        out_specs=[
            pl.BlockSpec((blk, _H), lambda i: (i, 0)),
            pl.BlockSpec((blk, _H), lambda i: (i, 0)),
            pl.BlockSpec((blk, _H), lambda i: (i, 0)),
        ],
        out_shape=[
            jax.ShapeDtypeStruct((_N, _H), jnp.float32),
            jax.ShapeDtypeStruct((_N, _H), jnp.float32),
            jax.ShapeDtypeStruct((_N, _H), jnp.float32),
        ],
    )(query, memory, Wq, bq.reshape(1, _H), Wk, bk.reshape(1, _H),
      Wv, bv.reshape(1, _H))


def _permute(a, idx):
    dnums = lax.GatherDimensionNumbers(
        offset_dims=(), collapsed_slice_dims=(0,), start_index_map=(0,))
    return lax.gather(a, idx[:, None], dnums, (1,),
                      mode=lax.GatherScatterMode.PROMISE_IN_BOUNDS)


_mesh = plsc.VectorSubcoreMesh(core_axis_name="c", subcore_axis_name="s")


@functools.partial(
    pl.kernel,
    out_type=jax.ShapeDtypeStruct((_NC, _N, _H), jnp.float32),
    mesh=_mesh,
    scratch_types=[
        pltpu.VMEM((_SLAB, _CH), jnp.int32),
        pltpu.VMEM((_SLAB, _CH), jnp.int32),
        pltpu.VMEM((_CH, _D), jnp.float32),
        pltpu.VMEM((_CH, _D), jnp.float32),
        pltpu.VMEM((_CH, _D), jnp.int32),
        pltpu.VMEM((_CH, _D), jnp.int32),
        pltpu.VMEM((_CH, _H), jnp.float32),
        pltpu.VMEM_SHARED((_N, _H), jnp.float32),
        pltpu.SemaphoreType.DMA,
        pltpu.SemaphoreType.DMA,
        pltpu.SemaphoreType.DMA,
    ],
)
def _edge_kernel(q_hbm, kv_hbm, row_hbm, col_hbm, out_hbm,
                 rsl, csl, qb0, qb1, kvb0, kvb1, msgb,
                 acc, gsem0, gsem1, ssem):
    c = lax.axis_index("c")
    s = lax.axis_index("s")
    wid = s * _NC + c
    iota = lax.iota(jnp.int32, _LANES)
    buf0 = (qb0, kvb0, gsem0)
    buf1 = (qb1, kvb1, gsem1)

    zero = jnp.zeros((_LANES,), jnp.float32)
    for r in range(_CH):
        for j in range(_H // _LANES):
            msgb[r, pl.ds(j * _LANES, _LANES)] = zero

    rows_per_tile = 624
    zbase = pl.multiple_of(s * rows_per_tile, 8)
    nfull = rows_per_tile // _CH
    rem = rows_per_tile - nfull * _CH
    for j in range(nfull):
        pltpu.sync_copy(msgb, acc.at[pl.ds(zbase + j * _CH, _CH)])
    if rem:
        pltpu.sync_copy(msgb.at[pl.ds(0, rem)],
                        acc.at[pl.ds(zbase + nfull * _CH, rem)])
    tail = _N - _NS * rows_per_tile

    @pl.when(s == _NS - 1)
    def _zero_tail():
        pltpu.sync_copy(msgb.at[pl.ds(0, tail)],
                        acc.at[pl.ds(_NS * rows_per_tile, tail)])

    plsc.subcore_barrier()

    def _fire(j, b):
        qb, kvb, gsem = b
        pltpu.async_copy(q_hbm.at[rsl.at[j]], qb, gsem)
        pltpu.async_copy(kv_hbm.at[csl.at[j]], kvb, gsem)

    def _wait_gather(j, b):
        qb, kvb, gsem = b
        pltpu.make_async_copy(q_hbm.at[rsl.at[j]], qb, gsem).wait()
        pltpu.make_async_copy(kv_hbm.at[csl.at[j]], kvb, gsem).wait()

    def _wait_scatter(j):
        pltpu.make_async_copy(msgb, acc.at[rsl.at[j]], ssem).wait()

    def _unpack(w):
        lo_u = w & 0xFFFF
        lo = ((lo_u ^ 0x8000) - 0x8000).astype(jnp.float32)
        hi = (w.astype(jnp.float32)
              - lo_u.astype(jnp.float32)) * (1.0 / 65536.0)
        return lo, hi

    def _compute_and_scatter(j, b):
        qb, kvb, _ = b
        for e in range(_CH):
            a = jnp.zeros((_LANES,), jnp.float32)
            for i in range(_W // _LANES):
                klo, khi = _unpack(kvb[e, pl.ds(i * _LANES, _LANES)])
                a = (a + qb[e, pl.ds(i * _LANES, _LANES)] * klo
                     + qb[e, pl.ds(_W + i * _LANES, _LANES)] * khi)
            for k in (8, 4, 2, 1):
                a = a + _permute(a, iota ^ k)
            cf = 1.0 / (1.0 + jnp.exp(a * (-1.0 / _SKV)))
            cf = cf * (1.0 / _SKV)
            for i in range(_W // _LANES):
                vlo, vhi = _unpack(kvb[e, pl.ds(_W + i * _LANES, _LANES)])
                msgb[e, pl.ds(i * _LANES, _LANES)] = vlo * cf
                msgb[e, pl.ds(_W + i * _LANES, _LANES)] = vhi * cf
        pltpu.async_copy(msgb, acc.at[rsl.at[j]], ssem, add=True)

    def _stage(j, cur, nxt):
        @pl.when(j + 1 < _SLAB)
        def _():
            _fire(j + 1, nxt)

        _wait_gather(j, cur)

        @pl.when(j >= 1)
        def _():
            _wait_scatter(j - 1)

        _compute_and_scatter(j, cur)

    def _slab(sl, carry):
        @pl.when(sl >= 1)
        def _():
            _wait_scatter(_SLAB - 1)

        pltpu.sync_copy(row_hbm.at[wid, sl], rsl)
        pltpu.sync_copy(col_hbm.at[wid, sl], csl)
        _fire(0, buf0)

        def _pair(p, cc):
            _stage(p * 2, buf0, buf1)
            _stage(p * 2 + 1, buf1, buf0)
            return cc

        lax.fori_loop(0, _SLAB // 2, _pair, 0)
        return carry

    lax.fori_loop(0, _NSLAB, _slab, 0)
    _wait_scatter(_SLAB - 1)

    plsc.subcore_barrier()

    pltpu.sync_copy(acc.at[pl.ds(zbase, rows_per_tile)],
                    out_hbm.at[c, pl.ds(zbase, rows_per_tile)])

    @pl.when(s == _NS - 1)
    def _write_tail():
        pltpu.sync_copy(acc.at[pl.ds(_NS * rows_per_tile, tail)],
                        out_hbm.at[c, pl.ds(_NS * rows_per_tile, tail)])


def _add_body(p_ref, o_ref):
    o_ref[...] = p_ref[0] + p_ref[1]


def _addp(partial):
    blk = 1000
    return pl.pallas_call(
        _add_body,
        grid=(_N // blk,),
        in_specs=[pl.BlockSpec((2, blk, _H), lambda i: (0, i, 0))],
        out_specs=pl.BlockSpec((blk, _H), lambda i: (i, 0)),
        out_shape=jax.ShapeDtypeStruct((_N, _H), jnp.float32),
    )(partial)


def kernel(query, memory, edge_index, Wq, bq, Wk, bk, Wv, bv):
    q, k, v = _qkv(query, memory, Wq, bq, Wk, bk, Wv, bv)
    kv = jnp.concatenate([_quant_pack(k, _SKV, 7.9),
                          _quant_pack(v, _SKV, 7.9)], axis=1)
    row = edge_index[0].reshape(_NW, _NSLAB, _SLAB, _CH)
    col = edge_index[1].reshape(_NW, _NSLAB, _SLAB, _CH)
    partial = _edge_kernel(q, kv, row, col)
    return _addp(partial)


# speedup vs baseline: 1.0129x; 1.0129x over previous
import functools

import numpy as np

import jax
import jax.numpy as jnp
from jax import lax
from jax.experimental import pallas as pl
from jax.experimental.pallas import tpu as pltpu
from jax.experimental.pallas import tpu_sc as plsc

_N = 10000
_D = 128
_H = 128
_NC = 2
_NS = 16
_NW = _NC * _NS
_E = 320000
_EPW = _E // _NW
_CH = 40
_NCH = _EPW // _CH
_SLAB = 50
_NSLAB = _NCH // _SLAB
_LANES = 16
_SCALE = 1.0 / float(_H) ** 0.5
_SQ = 32768.0
_SKV = 4096.0
_W = _D // 2


def _quant_pack(x, scale, lim):
    xi = jnp.round(jnp.clip(x, -lim, lim) * scale).astype(jnp.int32)
    lo = xi[:, :_W]
    lo_u = jnp.where(lo < 0, lo + 65536, lo)
    return lo_u + xi[:, _W:] * 65536


def _qkv_body(x_ref, m_ref, wq_ref, bq_ref, wk_ref, bk_ref, wv_ref, bv_ref,
              q_ref, k_ref, v_ref):
    x = x_ref[...]
    m = m_ref[...]
    q = jnp.dot(x, wq_ref[...], preferred_element_type=jnp.float32) + bq_ref[...]
    q_ref[...] = jax.nn.gelu(q) * _SCALE
    k = jnp.dot(m, wk_ref[...], preferred_element_type=jnp.float32) + bk_ref[...]
    v = jnp.dot(m, wv_ref[...], preferred_element_type=jnp.float32) + bv_ref[...]
    k_ref[...] = jax.nn.gelu(k)
    v_ref[...] = jax.nn.gelu(v)


def _qkv(query, memory, Wq, bq, Wk, bk, Wv, bv):
    blk = 1000
    return pl.pallas_call(
        _qkv_body,
        grid=(_N // blk,),
        in_specs=[
            pl.BlockSpec((blk, _D), lambda i: (i, 0)),
            pl.BlockSpec((blk, _D), lambda i: (i, 0)),
            pl.BlockSpec((_D, _H), lambda i: (0, 0)),
            pl.BlockSpec((1, _H), lambda i: (0, 0)),
            pl.BlockSpec((_D, _H), lambda i: (0, 0)),
            pl.BlockSpec((1, _H), lambda i: (0, 0)),
            pl.BlockSpec((_D, _H), lambda i: (0, 0)),
            pl.BlockSpec((1, _H), lambda i: (0, 0)),
        ],
        out_specs=[
            pl.BlockSpec((blk, _H), lambda i: (i, 0)),
            pl.BlockSpec((blk, _H), lambda i: (i, 0)),
            pl.BlockSpec((blk, _H), lambda i: (i, 0)),
        ],
        out_shape=[
            jax.ShapeDtypeStruct((_N, _H), jnp.float32),
            jax.ShapeDtypeStruct((_N, _H), jnp.float32),
            jax.ShapeDtypeStruct((_N, _H), jnp.float32),
        ],
    )(query, memory, Wq, bq.reshape(1, _H), Wk, bk.reshape(1, _H),
      Wv, bv.reshape(1, _H))


def _pack_body(k_ref, v_ref, kv_ref):
    kv_ref[...] = jnp.concatenate(
        [_quant_pack(k_ref[...], _SKV, 7.9),
         _quant_pack(v_ref[...], _SKV, 7.9)], axis=1)


def _pack(k, v):
    blk = 1000
    return pl.pallas_call(
        _pack_body,
        grid=(_N // blk,),
        in_specs=[
            pl.BlockSpec((blk, _H), lambda i: (i, 0)),
            pl.BlockSpec((blk, _H), lambda i: (i, 0)),
        ],
        out_specs=pl.BlockSpec((blk, _D), lambda i: (i, 0)),
        out_shape=jax.ShapeDtypeStruct((_N, _D), jnp.int32),
    )(k, v)


def _permute(a, idx):
    dnums = lax.GatherDimensionNumbers(
        offset_dims=(), collapsed_slice_dims=(0,), start_index_map=(0,))
    return lax.gather(a, idx[:, None], dnums, (1,),
                      mode=lax.GatherScatterMode.PROMISE_IN_BOUNDS)


_mesh = plsc.VectorSubcoreMesh(core_axis_name="c", subcore_axis_name="s")


@functools.partial(
    pl.kernel,
    out_type=jax.ShapeDtypeStruct((_NC, _N, _H), jnp.float32),
    mesh=_mesh,
    scratch_types=[
        pltpu.VMEM((_SLAB, _CH), jnp.int32),
        pltpu.VMEM((_SLAB, _CH), jnp.int32),
        pltpu.VMEM((_CH, _D), jnp.float32),
        pltpu.VMEM((_CH, _D), jnp.float32),
        pltpu.VMEM((_CH, _D), jnp.int32),
        pltpu.VMEM((_CH, _D), jnp.int32),
        pltpu.VMEM((_CH, _H), jnp.float32),
        pltpu.VMEM_SHARED((_N, _H), jnp.float32),
        pltpu.SemaphoreType.DMA,
        pltpu.SemaphoreType.DMA,
        pltpu.SemaphoreType.DMA,
    ],
)
def _edge_kernel(q_hbm, kv_hbm, row_hbm, col_hbm, out_hbm,
                 rsl, csl, qb0, qb1, kvb0, kvb1, msgb,
                 acc, gsem0, gsem1, ssem):
    c = lax.axis_index("c")
    s = lax.axis_index("s")
    wid = s * _NC + c
    iota = lax.iota(jnp.int32, _LANES)
    buf0 = (qb0, kvb0, gsem0)
    buf1 = (qb1, kvb1, gsem1)

    zero = jnp.zeros((_LANES,), jnp.float32)
    for r in range(_CH):
        for j in range(_H // _LANES):
            msgb[r, pl.ds(j * _LANES, _LANES)] = zero

    rows_per_tile = 624
    zbase = pl.multiple_of(s * rows_per_tile, 8)
    nfull = rows_per_tile // _CH
    rem = rows_per_tile - nfull * _CH
    for j in range(nfull):
        pltpu.sync_copy(msgb, acc.at[pl.ds(zbase + j * _CH, _CH)])
    if rem:
        pltpu.sync_copy(msgb.at[pl.ds(0, rem)],
                        acc.at[pl.ds(zbase + nfull * _CH, rem)])
    tail = _N - _NS * rows_per_tile

    @pl.when(s == _NS - 1)
    def _zero_tail():
        pltpu.sync_copy(msgb.at[pl.ds(0, tail)],
                        acc.at[pl.ds(_NS * rows_per_tile, tail)])

    plsc.subcore_barrier()

    def _fire(j, b):
        qb, kvb, gsem = b
        pltpu.async_copy(q_hbm.at[rsl.at[j]], qb, gsem)
        pltpu.async_copy(kv_hbm.at[csl.at[j]], kvb, gsem)

    def _wait_gather(j, b):
        qb, kvb, gsem = b
        pltpu.make_async_copy(q_hbm.at[rsl.at[j]], qb, gsem).wait()
        pltpu.make_async_copy(kv_hbm.at[csl.at[j]], kvb, gsem).wait()

    def _wait_scatter(j):
        pltpu.make_async_copy(msgb, acc.at[rsl.at[j]], ssem).wait()

    def _unpack(w):
        lo_u = w & 0xFFFF
        lo = ((lo_u ^ 0x8000) - 0x8000).astype(jnp.float32)
        hi = (w.astype(jnp.float32)
              - lo_u.astype(jnp.float32)) * (1.0 / 65536.0)
        return lo, hi

    def _compute_and_scatter(j, b):
        qb, kvb, _ = b
        for e in range(_CH):
            a = jnp.zeros((_LANES,), jnp.float32)
            for i in range(_W // _LANES):
                klo, khi = _unpack(kvb[e, pl.ds(i * _LANES, _LANES)])
                a = (a + qb[e, pl.ds(i * _LANES, _LANES)] * klo
                     + qb[e, pl.ds(_W + i * _LANES, _LANES)] * khi)
            for k in (8, 4, 2, 1):
                a = a + _permute(a, iota ^ k)
            cf = 1.0 / (1.0 + jnp.exp(a * (-1.0 / _SKV)))
            cf = cf * (1.0 / _SKV)
            for i in range(_W // _LANES):
                vlo, vhi = _unpack(kvb[e, pl.ds(_W + i * _LANES, _LANES)])
                msgb[e, pl.ds(i * _LANES, _LANES)] = vlo * cf
                msgb[e, pl.ds(_W + i * _LANES, _LANES)] = vhi * cf
        pltpu.async_copy(msgb, acc.at[rsl.at[j]], ssem, add=True)

    def _stage(j, cur, nxt):
        @pl.when(j + 1 < _SLAB)
        def _():
            _fire(j + 1, nxt)

        _wait_gather(j, cur)

        @pl.when(j >= 1)
        def _():
            _wait_scatter(j - 1)

        _compute_and_scatter(j, cur)

    def _slab(sl, carry):
        @pl.when(sl >= 1)
        def _():
            _wait_scatter(_SLAB - 1)

        pltpu.sync_copy(row_hbm.at[wid, sl], rsl)
        pltpu.sync_copy(col_hbm.at[wid, sl], csl)
        _fire(0, buf0)

        def _pair(p, cc):
            _stage(p * 2, buf0, buf1)
            _stage(p * 2 + 1, buf1, buf0)
            return cc

        lax.fori_loop(0, _SLAB // 2, _pair, 0)
        return carry

    lax.fori_loop(0, _NSLAB, _slab, 0)
    _wait_scatter(_SLAB - 1)

    plsc.subcore_barrier()

    pltpu.sync_copy(acc.at[pl.ds(zbase, rows_per_tile)],
                    out_hbm.at[c, pl.ds(zbase, rows_per_tile)])

    @pl.when(s == _NS - 1)
    def _write_tail():
        pltpu.sync_copy(acc.at[pl.ds(_NS * rows_per_tile, tail)],
                        out_hbm.at[c, pl.ds(_NS * rows_per_tile, tail)])


def _add_body(p_ref, o_ref):
    o_ref[...] = p_ref[0] + p_ref[1]


def _addp(partial):
    blk = 1000
    return pl.pallas_call(
        _add_body,
        grid=(_N // blk,),
        in_specs=[pl.BlockSpec((2, blk, _H), lambda i: (0, i, 0))],
        out_specs=pl.BlockSpec((blk, _H), lambda i: (i, 0)),
        out_shape=jax.ShapeDtypeStruct((_N, _H), jnp.float32),
    )(partial)


def kernel(query, memory, edge_index, Wq, bq, Wk, bk, Wv, bv):
    q, k, v = _qkv(query, memory, Wq, bq, Wk, bk, Wv, bv)
    kv = _pack(k, v)
    row = edge_index[0].reshape(_NW, _NSLAB, _SLAB, _CH)
    col = edge_index[1].reshape(_NW, _NSLAB, _SLAB, _CH)
    partial = _edge_kernel(q, kv, row, col)
    return _addp(partial)


# speedup vs baseline: 1.0205x; 1.0075x over previous
import functools

import numpy as np

import jax
import jax.numpy as jnp
from jax import lax
from jax.experimental import pallas as pl
from jax.experimental.pallas import tpu as pltpu
from jax.experimental.pallas import tpu_sc as plsc

_N = 10000
_D = 128
_H = 128
_NC = 2
_NS = 16
_NW = _NC * _NS
_E = 320000
_EPW = _E // _NW
_CH = 40
_NCH = _EPW // _CH
_SLAB = 50
_NSLAB = _NCH // _SLAB
_LANES = 16
_SCALE = 1.0 / float(_H) ** 0.5
_SQ = 32768.0
_SKV = 4096.0
_W = _D // 2


def _quant_pack(x, scale, lim):
    xi = jnp.round(jnp.clip(x, -lim, lim) * scale).astype(jnp.int32)
    lo = xi[:, :_W]
    lo_u = jnp.where(lo < 0, lo + 65536, lo)
    return lo_u + xi[:, _W:] * 65536


def _qkv_body(x_ref, m_ref, wq_ref, bq_ref, wk_ref, bk_ref, wv_ref, bv_ref,
              q_ref, kv_ref):
    x = x_ref[...]
    m = m_ref[...]
    q = jnp.dot(x, wq_ref[...], preferred_element_type=jnp.float32) + bq_ref[...]
    q_ref[...] = jax.nn.gelu(q) * _SCALE
    k = jnp.dot(m, wk_ref[...], preferred_element_type=jnp.float32) + bk_ref[...]
    v = jnp.dot(m, wv_ref[...], preferred_element_type=jnp.float32) + bv_ref[...]
    kv_ref[...] = jnp.concatenate(
        [_quant_pack(jax.nn.gelu(k), _SKV, 7.9),
         _quant_pack(jax.nn.gelu(v), _SKV, 7.9)], axis=1)


def _qkv(query, memory, Wq, bq, Wk, bk, Wv, bv):
    blk = 1000
    return pl.pallas_call(
        _qkv_body,
        grid=(_N // blk,),
        in_specs=[
            pl.BlockSpec((blk, _D), lambda i: (i, 0)),
            pl.BlockSpec((blk, _D), lambda i: (i, 0)),
            pl.BlockSpec((_D, _H), lambda i: (0, 0)),
            pl.BlockSpec((1, _H), lambda i: (0, 0)),
            pl.BlockSpec((_D, _H), lambda i: (0, 0)),
            pl.BlockSpec((1, _H), lambda i: (0, 0)),
            pl.BlockSpec((_D, _H), lambda i: (0, 0)),
            pl.BlockSpec((1, _H), lambda i: (0, 0)),
        ],
        out_specs=[
            pl.BlockSpec((blk, _H), lambda i: (i, 0)),
            pl.BlockSpec((blk, _D), lambda i: (i, 0)),
        ],
        out_shape=[
            jax.ShapeDtypeStruct((_N, _H), jnp.float32),
            jax.ShapeDtypeStruct((_N, _D), jnp.int32),
        ],
    )(query, memory, Wq, bq.reshape(1, _H), Wk, bk.reshape(1, _H),
      Wv, bv.reshape(1, _H))


def _permute(a, idx):
    dnums = lax.GatherDimensionNumbers(
        offset_dims=(), collapsed_slice_dims=(0,), start_index_map=(0,))
    return lax.gather(a, idx[:, None], dnums, (1,),
                      mode=lax.GatherScatterMode.PROMISE_IN_BOUNDS)


_mesh = plsc.VectorSubcoreMesh(core_axis_name="c", subcore_axis_name="s")


@functools.partial(
    pl.kernel,
    out_type=jax.ShapeDtypeStruct((_NC, _N, _H), jnp.float32),
    mesh=_mesh,
    scratch_types=[
        pltpu.VMEM((_SLAB, _CH), jnp.int32),
        pltpu.VMEM((_SLAB, _CH), jnp.int32),
        pltpu.VMEM((_CH, _D), jnp.float32),
        pltpu.VMEM((_CH, _D), jnp.float32),
        pltpu.VMEM((_CH, _D), jnp.int32),
        pltpu.VMEM((_CH, _D), jnp.int32),
        pltpu.VMEM((_CH, _H), jnp.float32),
        pltpu.VMEM_SHARED((_N, _H), jnp.float32),
        pltpu.SemaphoreType.DMA,
        pltpu.SemaphoreType.DMA,
        pltpu.SemaphoreType.DMA,
    ],
)
def _edge_kernel(q_hbm, kv_hbm, row_hbm, col_hbm, out_hbm,
                 rsl, csl, qb0, qb1, kvb0, kvb1, msgb,
                 acc, gsem0, gsem1, ssem):
    c = lax.axis_index("c")
    s = lax.axis_index("s")
    wid = s * _NC + c
    iota = lax.iota(jnp.int32, _LANES)
    buf0 = (qb0, kvb0, gsem0)
    buf1 = (qb1, kvb1, gsem1)

    zero = jnp.zeros((_LANES,), jnp.float32)
    for r in range(_CH):
        for j in range(_H // _LANES):
            msgb[r, pl.ds(j * _LANES, _LANES)] = zero

    rows_per_tile = 624
    zbase = pl.multiple_of(s * rows_per_tile, 8)
    nfull = rows_per_tile // _CH
    rem = rows_per_tile - nfull * _CH
    for j in range(nfull):
        pltpu.sync_copy(msgb, acc.at[pl.ds(zbase + j * _CH, _CH)])
    if rem:
        pltpu.sync_copy(msgb.at[pl.ds(0, rem)],
                        acc.at[pl.ds(zbase + nfull * _CH, rem)])
    tail = _N - _NS * rows_per_tile

    @pl.when(s == _NS - 1)
    def _zero_tail():
        pltpu.sync_copy(msgb.at[pl.ds(0, tail)],
                        acc.at[pl.ds(_NS * rows_per_tile, tail)])

    plsc.subcore_barrier()

    def _fire(j, b):
        qb, kvb, gsem = b
        pltpu.async_copy(q_hbm.at[rsl.at[j]], qb, gsem)
        pltpu.async_copy(kv_hbm.at[csl.at[j]], kvb, gsem)

    def _wait_gather(j, b):
        qb, kvb, gsem = b
        pltpu.make_async_copy(q_hbm.at[rsl.at[j]], qb, gsem).wait()
        pltpu.make_async_copy(kv_hbm.at[csl.at[j]], kvb, gsem).wait()

    def _wait_scatter(j):
        pltpu.make_async_copy(msgb, acc.at[rsl.at[j]], ssem).wait()

    def _unpack(w):
        lo_u = w & 0xFFFF
        lo = ((lo_u ^ 0x8000) - 0x8000).astype(jnp.float32)
        hi = (w.astype(jnp.float32)
              - lo_u.astype(jnp.float32)) * (1.0 / 65536.0)
        return lo, hi

    def _compute_and_scatter(j, b):
        qb, kvb, _ = b
        for e in range(_CH):
            a = jnp.zeros((_LANES,), jnp.float32)
            for i in range(_W // _LANES):
                klo, khi = _unpack(kvb[e, pl.ds(i * _LANES, _LANES)])
                a = (a + qb[e, pl.ds(i * _LANES, _LANES)] * klo
                     + qb[e, pl.ds(_W + i * _LANES, _LANES)] * khi)
            for k in (8, 4, 2, 1):
                a = a + _permute(a, iota ^ k)
            cf = 1.0 / (1.0 + jnp.exp(a * (-1.0 / _SKV)))
            cf = cf * (1.0 / _SKV)
            for i in range(_W // _LANES):
                vlo, vhi = _unpack(kvb[e, pl.ds(_W + i * _LANES, _LANES)])
                msgb[e, pl.ds(i * _LANES, _LANES)] = vlo * cf
                msgb[e, pl.ds(_W + i * _LANES, _LANES)] = vhi * cf
        pltpu.async_copy(msgb, acc.at[rsl.at[j]], ssem, add=True)

    def _stage(j, cur, nxt):
        @pl.when(j + 1 < _SLAB)
        def _():
            _fire(j + 1, nxt)

        _wait_gather(j, cur)

        @pl.when(j >= 1)
        def _():
            _wait_scatter(j - 1)

        _compute_and_scatter(j, cur)

    def _slab(sl, carry):
        @pl.when(sl >= 1)
        def _():
            _wait_scatter(_SLAB - 1)

        pltpu.sync_copy(row_hbm.at[wid, sl], rsl)
        pltpu.sync_copy(col_hbm.at[wid, sl], csl)
        _fire(0, buf0)

        def _pair(p, cc):
            _stage(p * 2, buf0, buf1)
            _stage(p * 2 + 1, buf1, buf0)
            return cc

        lax.fori_loop(0, _SLAB // 2, _pair, 0)
        return carry

    lax.fori_loop(0, _NSLAB, _slab, 0)
    _wait_scatter(_SLAB - 1)

    plsc.subcore_barrier()

    pltpu.sync_copy(acc.at[pl.ds(zbase, rows_per_tile)],
                    out_hbm.at[c, pl.ds(zbase, rows_per_tile)])

    @pl.when(s == _NS - 1)
    def _write_tail():
        pltpu.sync_copy(acc.at[pl.ds(_NS * rows_per_tile, tail)],
                        out_hbm.at[c, pl.ds(_NS * rows_per_tile, tail)])


def _add_body(p_ref, o_ref):
    o_ref[...] = p_ref[0] + p_ref[1]


def _addp(partial):
    blk = 1000
    return pl.pallas_call(
        _add_body,
        grid=(_N // blk,),
        in_specs=[pl.BlockSpec((2, blk, _H), lambda i: (0, i, 0))],
        out_specs=pl.BlockSpec((blk, _H), lambda i: (i, 0)),
        out_shape=jax.ShapeDtypeStruct((_N, _H), jnp.float32),
    )(partial)


def kernel(query, memory, edge_index, Wq, bq, Wk, bk, Wv, bv):
    q, kv = _qkv(query, memory, Wq, bq, Wk, bk, Wv, bv)
    row = edge_index[0].reshape(_NW, _NSLAB, _SLAB, _CH)
    col = edge_index[1].reshape(_NW, _NSLAB, _SLAB, _CH)
    partial = _edge_kernel(q, kv, row, col)
    return _addp(partial)


# speedup vs baseline: 1.3381x; 1.3112x over previous
import functools

import numpy as np

import jax
import jax.numpy as jnp
from jax import lax
from jax.experimental import pallas as pl
from jax.experimental.pallas import tpu as pltpu
from jax.experimental.pallas import tpu_sc as plsc

_N = 10000
_D = 128
_H = 128
_NC = 2
_NS = 16
_NW = _NC * _NS
_E = 320000
_EPW = _E // _NW
_CH = 40
_NCH = _EPW // _CH
_SLAB = 50
_NSLAB = _NCH // _SLAB
_LANES = 16
_SCALE = 1.0 / float(_H) ** 0.5
_SQ = 32768.0
_SKV = 4096.0
_W = _D // 2


def _quant_pack(x, scale, lim):
    xi = jnp.round(jnp.clip(x, -lim, lim) * scale).astype(jnp.int32)
    lo = xi[:, :_W]
    lo_u = jnp.where(lo < 0, lo + 65536, lo)
    return lo_u + xi[:, _W:] * 65536


def _qkv_body(x_ref, m_ref, wq_ref, bq_ref, wk_ref, bk_ref, wv_ref, bv_ref,
              q_ref, kv_ref):
    x = x_ref[...]
    m = m_ref[...]
    q = jnp.dot(x, wq_ref[...], preferred_element_type=jnp.float32) + bq_ref[...]
    q_ref[...] = jax.nn.gelu(q) * _SCALE
    k = jnp.dot(m, wk_ref[...], preferred_element_type=jnp.float32) + bk_ref[...]
    v = jnp.dot(m, wv_ref[...], preferred_element_type=jnp.float32) + bv_ref[...]
    kv_ref[...] = jnp.concatenate(
        [_quant_pack(jax.nn.gelu(k), _SKV, 7.9),
         _quant_pack(jax.nn.gelu(v), _SKV, 7.9)], axis=1)


def _qkv(query, memory, Wq, bq, Wk, bk, Wv, bv):
    blk = 1000
    return pl.pallas_call(
        _qkv_body,
        grid=(_N // blk,),
        in_specs=[
            pl.BlockSpec((blk, _D), lambda i: (i, 0)),
            pl.BlockSpec((blk, _D), lambda i: (i, 0)),
            pl.BlockSpec((_D, _H), lambda i: (0, 0)),
            pl.BlockSpec((1, _H), lambda i: (0, 0)),
            pl.BlockSpec((_D, _H), lambda i: (0, 0)),
            pl.BlockSpec((1, _H), lambda i: (0, 0)),
            pl.BlockSpec((_D, _H), lambda i: (0, 0)),
            pl.BlockSpec((1, _H), lambda i: (0, 0)),
        ],
        out_specs=[
            pl.BlockSpec((blk, _H), lambda i: (i, 0)),
            pl.BlockSpec((blk, _D), lambda i: (i, 0)),
        ],
        out_shape=[
            jax.ShapeDtypeStruct((_N, _H), jnp.float32),
            jax.ShapeDtypeStruct((_N, _D), jnp.int32),
        ],
    )(query, memory, Wq, bq.reshape(1, _H), Wk, bk.reshape(1, _H),
      Wv, bv.reshape(1, _H))


def _permute(a, idx):
    dnums = lax.GatherDimensionNumbers(
        offset_dims=(), collapsed_slice_dims=(0,), start_index_map=(0,))
    return lax.gather(a, idx[:, None], dnums, (1,),
                      mode=lax.GatherScatterMode.PROMISE_IN_BOUNDS)


_mesh = plsc.VectorSubcoreMesh(core_axis_name="c", subcore_axis_name="s")


@functools.partial(
    pl.kernel,
    out_type=jax.ShapeDtypeStruct((_NC, _N, _H), jnp.float32),
    mesh=_mesh,
    scratch_types=[
        pltpu.VMEM((_SLAB, _CH), jnp.int32),
        pltpu.VMEM((_SLAB, _CH), jnp.int32),
        pltpu.VMEM((_CH, _D), jnp.float32),
        pltpu.VMEM((_CH, _D), jnp.float32),
        pltpu.VMEM((_CH, _D), jnp.int32),
        pltpu.VMEM((_CH, _D), jnp.int32),
        pltpu.VMEM((_CH, _H), jnp.float32),
        pltpu.VMEM_SHARED((_N, _H), jnp.float32),
        pltpu.SemaphoreType.DMA,
        pltpu.SemaphoreType.DMA,
        pltpu.SemaphoreType.DMA,
    ],
)
def _edge_kernel(q_hbm, kv_hbm, row_hbm, col_hbm, out_hbm,
                 rsl, csl, qb0, qb1, kvb0, kvb1, msgb,
                 acc, gsem0, gsem1, ssem):
    c = lax.axis_index("c")
    s = lax.axis_index("s")
    wid = s * _NC + c
    iota = lax.iota(jnp.int32, _LANES)
    buf0 = (qb0, kvb0, gsem0)
    buf1 = (qb1, kvb1, gsem1)

    zero = jnp.zeros((_LANES,), jnp.float32)
    for r in range(_CH):
        for j in range(_H // _LANES):
            msgb[r, pl.ds(j * _LANES, _LANES)] = zero

    rows_per_tile = 624
    zbase = pl.multiple_of(s * rows_per_tile, 8)
    nfull = rows_per_tile // _CH
    rem = rows_per_tile - nfull * _CH
    for j in range(nfull):
        pltpu.sync_copy(msgb, acc.at[pl.ds(zbase + j * _CH, _CH)])
    if rem:
        pltpu.sync_copy(msgb.at[pl.ds(0, rem)],
                        acc.at[pl.ds(zbase + nfull * _CH, rem)])
    tail = _N - _NS * rows_per_tile

    @pl.when(s == _NS - 1)
    def _zero_tail():
        pltpu.sync_copy(msgb.at[pl.ds(0, tail)],
                        acc.at[pl.ds(_NS * rows_per_tile, tail)])

    plsc.subcore_barrier()

    def _fire(j, b):
        qb, kvb, gsem = b
        pltpu.async_copy(q_hbm.at[rsl.at[j]], qb, gsem)
        pltpu.async_copy(kv_hbm.at[csl.at[j]], kvb, gsem)

    def _wait_gather(j, b):
        qb, kvb, gsem = b
        pltpu.make_async_copy(q_hbm.at[rsl.at[j]], qb, gsem).wait()
        pltpu.make_async_copy(kv_hbm.at[csl.at[j]], kvb, gsem).wait()

    def _wait_scatter(j):
        pltpu.make_async_copy(msgb, acc.at[rsl.at[j]], ssem).wait()

    def _unpack(w):
        lo = ((w << 16) >> 16).astype(jnp.float32)
        hi = (w >> 16).astype(jnp.float32)
        return lo, hi

    def _compute_and_scatter(j, b):
        qb, kvb, _ = b
        for e in range(_CH):
            a = jnp.zeros((_LANES,), jnp.float32)
            for i in range(_W // _LANES):
                klo, khi = _unpack(kvb[e, pl.ds(i * _LANES, _LANES)])
                a = (a + qb[e, pl.ds(i * _LANES, _LANES)] * klo
                     + qb[e, pl.ds(_W + i * _LANES, _LANES)] * khi)
            for k in (8, 4, 2, 1):
                a = a + _permute(a, iota ^ k)
            cf = 1.0 / (1.0 + jnp.exp(a * (-1.0 / _SKV)))
            cf = cf * (1.0 / _SKV)
            for i in range(_W // _LANES):
                vlo, vhi = _unpack(kvb[e, pl.ds(_W + i * _LANES, _LANES)])
                msgb[e, pl.ds(i * _LANES, _LANES)] = vlo * cf
                msgb[e, pl.ds(_W + i * _LANES, _LANES)] = vhi * cf
        pltpu.async_copy(msgb, acc.at[rsl.at[j]], ssem, add=True)

    def _stage(j, cur, nxt):
        @pl.when(j + 1 < _SLAB)
        def _():
            _fire(j + 1, nxt)

        _wait_gather(j, cur)

        @pl.when(j >= 1)
        def _():
            _wait_scatter(j - 1)

        _compute_and_scatter(j, cur)

    def _slab(sl, carry):
        @pl.when(sl >= 1)
        def _():
            _wait_scatter(_SLAB - 1)

        pltpu.sync_copy(row_hbm.at[wid, sl], rsl)
        pltpu.sync_copy(col_hbm.at[wid, sl], csl)
        _fire(0, buf0)

        def _pair(p, cc):
            _stage(p * 2, buf0, buf1)
            _stage(p * 2 + 1, buf1, buf0)
            return cc

        lax.fori_loop(0, _SLAB // 2, _pair, 0)
        return carry

    lax.fori_loop(0, _NSLAB, _slab, 0)
    _wait_scatter(_SLAB - 1)

    plsc.subcore_barrier()

    pltpu.sync_copy(acc.at[pl.ds(zbase, rows_per_tile)],
                    out_hbm.at[c, pl.ds(zbase, rows_per_tile)])

    @pl.when(s == _NS - 1)
    def _write_tail():
        pltpu.sync_copy(acc.at[pl.ds(_NS * rows_per_tile, tail)],
                        out_hbm.at[c, pl.ds(_NS * rows_per_tile, tail)])


def _add_body(p_ref, o_ref):
    o_ref[...] = p_ref[0] + p_ref[1]


def _addp(partial):
    blk = 1000
    return pl.pallas_call(
        _add_body,
        grid=(_N // blk,),
        in_specs=[pl.BlockSpec((2, blk, _H), lambda i: (0, i, 0))],
        out_specs=pl.BlockSpec((blk, _H), lambda i: (i, 0)),
        out_shape=jax.ShapeDtypeStruct((_N, _H), jnp.float32),
    )(partial)


def kernel(query, memory, edge_index, Wq, bq, Wk, bk, Wv, bv):
    q, kv = _qkv(query, memory, Wq, bq, Wk, bk, Wv, bv)
    row = edge_index[0].reshape(_NW, _NSLAB, _SLAB, _CH)
    col = edge_index[1].reshape(_NW, _NSLAB, _SLAB, _CH)
    partial = _edge_kernel(q, kv, row, col)
    return _addp(partial)
